# Initial kernel scaffold; baseline (speedup 1.0000x reference)
import jax
import jax.numpy as jnp
from jax.experimental import pallas as pl


def kernel(seq, seg, tok_table, seg_table, pos_table):
    raise NotImplementedError("write your pallas kernel here")


# speedup vs baseline: 4.5147x; 4.5147x over previous
import functools

import jax
import jax.numpy as jnp
from jax import lax
from jax.experimental import pallas as pl
from jax.experimental.pallas import tpu as pltpu
from jax.experimental.pallas import tpu_sc as plsc

B = 4096
L = 200
D = 64
N = B * L
NW = 32
NPW = N // NW
C = 640
SUB = 128
NSUB = C // SUB
NCHUNK = NPW // C
LANES = 16
DV = D // LANES


def _embed_kernel(seq_hbm, seg_hbm, tok_hbm, segt_hbm, post_hbm, out_hbm,
                  seg_v, pos_v, comb_v, sidx_v, cidx_v, buf_v, sem):
    wid = lax.axis_index("s") * 2 + lax.axis_index("c")
    base = wid * NPW

    pltpu.sync_copy(segt_hbm, seg_v)
    pltpu.sync_copy(post_hbm, pos_v)

    def build_comb(l, _):
        for j in range(DV):
            sl = pl.ds(j * LANES, LANES)
            p = pos_v[l, sl]
            comb_v[l, sl] = p + seg_v[0, sl]
            comb_v[l + L, sl] = p + seg_v[1, sl]
        return 0
    lax.fori_loop(0, L, build_comb, 0)

    lane = lax.broadcasted_iota(jnp.int32, (LANES,), 0)

    def chunk(t, _):
        off = base + t * C
        pltpu.sync_copy(seq_hbm.at[pl.ds(off, C)], sidx_v)
        pltpu.sync_copy(seg_hbm.at[pl.ds(off, C)], cidx_v)

        def cfix(g, _):
            sl = pl.ds(g * LANES, LANES)
            n_vec = (off + g * LANES) + lane
            cidx_v[sl] = cidx_v[sl] * L + n_vec % L
            return 0
        lax.fori_loop(0, C // LANES, cfix, 0)

        copies = []
        for k in range(NSUB):
            sl = pl.ds(k * SUB, SUB)
            copies.append(
                pltpu.async_copy(tok_hbm.at[sidx_v.at[sl]], buf_v.at[sl], sem))
        for cp in copies:
            cp.wait()

        def add16(g, _):
            r0 = g * LANES
            cvec = cidx_v[pl.ds(r0, LANES)]
            for i in range(LANES):
                ci = cvec[i]
                for j in range(DV):
                    sl = pl.ds(j * LANES, LANES)
                    buf_v[r0 + i, sl] = buf_v[r0 + i, sl] + comb_v[ci, sl]
            return 0
        lax.fori_loop(0, C // LANES, add16, 0)

        pltpu.sync_copy(buf_v, out_hbm.at[pl.ds(off, C)])
        return 0

    lax.fori_loop(0, NCHUNK, chunk, 0)


@functools.partial(jax.jit, static_argnames=())
def kernel(seq, seg, tok_table, seg_table, pos_table):
    seq_flat = seq.reshape(N).astype(jnp.int32)
    seg_flat = seg.reshape(N).astype(jnp.int32)
    mesh = plsc.VectorSubcoreMesh(core_axis_name="c", subcore_axis_name="s")
    run = pl.kernel(
        _embed_kernel,
        mesh=mesh,
        compiler_params=pltpu.CompilerParams(use_tc_tiling_on_sc=False),
        out_type=jax.ShapeDtypeStruct((N, D), jnp.float32),
        scratch_types=[
            pltpu.VMEM((2, D), jnp.float32),
            pltpu.VMEM((L, D), jnp.float32),
            pltpu.VMEM((2 * L, D), jnp.float32),
            pltpu.VMEM((C,), jnp.int32),
            pltpu.VMEM((C,), jnp.int32),
            pltpu.VMEM((C, D), jnp.float32),
            pltpu.SemaphoreType.DMA,
        ],
    )
    out = run(seq_flat, seg_flat, tok_table, seg_table, pos_table)
    return out.reshape(B, L, D)


# speedup vs baseline: 7.1925x; 1.5931x over previous
import functools

import jax
import jax.numpy as jnp
from jax import lax
from jax.experimental import pallas as pl
from jax.experimental.pallas import tpu as pltpu
from jax.experimental.pallas import tpu_sc as plsc

B = 4096
L = 200
D = 64
N = B * L
NW = 32
NPW = N // NW
C = 512
SUB = 128
NSUB = C // SUB
NCHUNK = NPW // C
LANES = 16
DV = D // LANES


def _embed_kernel(seq_hbm, seg_hbm, tok_hbm, segt_hbm, post_hbm, out_hbm,
                  seg_v, pos_v, comb_v, sidx_v, cidx_v, buf_v, gsems, ssems):
    wid = lax.axis_index("s") * 2 + lax.axis_index("c")
    base = wid * NPW

    pltpu.sync_copy(segt_hbm, seg_v)
    pltpu.sync_copy(post_hbm, pos_v)

    sls = [pl.ds(j * LANES, LANES) for j in range(DV)]

    @plsc.parallel_loop(0, L, unroll=2)
    def build_comb(l):
        p = [pos_v[l, sl] for sl in sls]
        s0 = [seg_v[0, sl] for sl in sls]
        s1 = [seg_v[1, sl] for sl in sls]
        for j in range(DV):
            comb_v[l, sls[j]] = p[j] + s0[j]
        for j in range(DV):
            comb_v[l + L, sls[j]] = p[j] + s1[j]

    lane = lax.broadcasted_iota(jnp.int32, (LANES,), 0)

    def prep(t, s):
        off = base + t * C
        pltpu.sync_copy(seq_hbm.at[pl.ds(off, C)], sidx_v.at[s])
        pltpu.sync_copy(seg_hbm.at[pl.ds(off, C)], cidx_v.at[s])

        @plsc.parallel_loop(0, C // LANES, unroll=2)
        def cfix(g):
            sl = pl.ds(g * LANES, LANES)
            n_vec = (off + g * LANES) + lane
            cidx_v[s, sl] = cidx_v[s, sl] * L + n_vec % L

        for k in range(NSUB):
            sl = pl.ds(k * SUB, SUB)
            pltpu.async_copy(tok_hbm.at[sidx_v.at[s, sl]],
                             buf_v.at[s, sl], gsems.at[s])

    def work_add(s):
        pltpu.make_async_copy(out_hbm.at[pl.ds(0, C)], buf_v.at[s],
                              gsems.at[s]).wait()

        @plsc.parallel_loop(0, C // LANES, unroll=2)
        def add16(g):
            r0 = g * LANES
            cvec = cidx_v[s, pl.ds(r0, LANES)]
            nxt = [comb_v[cvec[0], sl] for sl in sls]
            for i in range(LANES):
                cur = nxt
                if i + 1 < LANES:
                    nxt = [comb_v[cvec[i + 1], sl] for sl in sls]
                for j in range(DV):
                    plsc.addupdate(buf_v.at[s, r0 + i, sls[j]], cur[j])

    def store_fire(t, s):
        off = base + t * C
        pltpu.async_copy(buf_v.at[s], out_hbm.at[pl.ds(off, C)], ssems.at[s])

    def drain_store(s):
        pltpu.make_async_copy(buf_v.at[s], out_hbm.at[pl.ds(0, C)],
                              ssems.at[s]).wait()

    prep(0, 0)

    def pair(i, _):
        @pl.when(i >= 1)
        def _():
            drain_store(1)
        prep(2 * i + 1, 1)
        work_add(0)
        store_fire(2 * i, 0)
        work_add(1)
        drain_store(0)

        @pl.when(i < NCHUNK // 2 - 1)
        def _():
            prep(2 * i + 2, 0)
        store_fire(2 * i + 1, 1)
        return 0

    lax.fori_loop(0, NCHUNK // 2, pair, 0)
    drain_store(1)


def kernel(seq, seg, tok_table, seg_table, pos_table):
    seq_flat = seq.reshape(N).astype(jnp.int32)
    seg_flat = seg.reshape(N).astype(jnp.int32)
    mesh = plsc.VectorSubcoreMesh(core_axis_name="c", subcore_axis_name="s")
    run = pl.kernel(
        _embed_kernel,
        mesh=mesh,
        compiler_params=pltpu.CompilerParams(use_tc_tiling_on_sc=False),
        out_type=jax.ShapeDtypeStruct((N, D), jnp.float32),
        scratch_types=[
            pltpu.VMEM((2, D), jnp.float32),
            pltpu.VMEM((L, D), jnp.float32),
            pltpu.VMEM((2 * L, D), jnp.float32),
            pltpu.VMEM((2, C), jnp.int32),
            pltpu.VMEM((2, C), jnp.int32),
            pltpu.VMEM((2, C, D), jnp.float32),
            pltpu.SemaphoreType.DMA((2,)),
            pltpu.SemaphoreType.DMA((2,)),
        ],
    )
    out = run(seq_flat, seg_flat, tok_table, seg_table, pos_table)
    return out.reshape(B, L, D)


# speedup vs baseline: 7.4510x; 1.0359x over previous
import jax
import jax.numpy as jnp
from jax import lax
from jax.experimental import pallas as pl
from jax.experimental.pallas import tpu as pltpu
from jax.experimental.pallas import tpu_sc as plsc

B = 4096
L = 200
D = 64
N = B * L
NW = 32
NPW = N // NW
C = 320
GSPLIT = (128, 128, 64)
NCHUNK = NPW // C
LANES = 16
DV = D // LANES


def _embed_kernel(seq_hbm, seg_hbm, tok_hbm, segt_hbm, post_hbm, out_hbm,
                  seg_v, pos_v, comb_v, sidx_v, cidx_v, buf_v, obuf_v,
                  gsems, ssems, isems):
    wid = lax.axis_index("s") * 2 + lax.axis_index("c")
    base = wid * NPW

    pltpu.sync_copy(segt_hbm, seg_v)
    pltpu.sync_copy(post_hbm, pos_v)

    sls = [pl.ds(j * LANES, LANES) for j in range(DV)]

    @plsc.parallel_loop(0, L, unroll=2)
    def build_comb(l):
        p = [pos_v[l, sl] for sl in sls]
        s0 = [seg_v[0, sl] for sl in sls]
        s1 = [seg_v[1, sl] for sl in sls]
        for j in range(DV):
            comb_v[l, sls[j]] = p[j] + s0[j]
        for j in range(DV):
            comb_v[l + L, sls[j]] = p[j] + s1[j]

    lane = lax.broadcasted_iota(jnp.int32, (LANES,), 0)


    def fire_idx(t, ip):
        off = base + t * C
        pltpu.async_copy(seq_hbm.at[pl.ds(off, C)], sidx_v.at[ip], isems.at[ip])
        pltpu.async_copy(seg_hbm.at[pl.ds(off, C)], cidx_v.at[ip], isems.at[ip])

    def drain_idx(ip):
        pltpu.make_async_copy(seq_hbm.at[pl.ds(0, C)], sidx_v.at[ip],
                              isems.at[ip]).wait()
        pltpu.make_async_copy(seg_hbm.at[pl.ds(0, C)], cidx_v.at[ip],
                              isems.at[ip]).wait()

    def drain_store(s):
        pltpu.make_async_copy(obuf_v.at[s], out_hbm.at[pl.ds(0, C * D)],
                              ssems.at[s]).wait()

    def gfire(t, ip, s, drain_st):
        off = base + t * C
        drain_idx(ip)

        @plsc.parallel_loop(0, C // LANES, unroll=2)
        def cfix(g):
            sl = pl.ds(g * LANES, LANES)
            n_vec = (off + g * LANES) + lane
            cidx_v[ip, sl] = cidx_v[ip, sl] * L + n_vec % L

        if isinstance(drain_st, bool):
            if drain_st:
                drain_store(s)
        else:
            @pl.when(drain_st)
            def _():
                drain_store(s)
        o = 0
        for n in GSPLIT:
            sl = pl.ds(o, n)
            pltpu.async_copy(tok_hbm.at[sidx_v.at[ip, sl]],
                             buf_v.at[s, sl], gsems.at[s])
            o += n

    def work_add(ip, s):
        pltpu.make_async_copy(tok_hbm.at[pl.ds(0, C)],
                              buf_v.at[s], gsems.at[s]).wait()

        @plsc.parallel_loop(0, C // LANES, unroll=2)
        def add16(g):
            r0 = g * LANES
            cvec = cidx_v[ip, pl.ds(r0, LANES)]

            def loads(i):
                ci = cvec[i]
                return ([comb_v[ci, sl] for sl in sls],
                        [buf_v[s, r0 + i, sl] for sl in sls])

            nxt = loads(0)
            for i in range(LANES):
                curc, curt = nxt
                if i + 1 < LANES:
                    nxt = loads(i + 1)
                ob = (r0 + i) * D
                for j in range(DV):
                    obuf_v[s, pl.ds(ob + j * LANES, LANES)] = (
                        curt[j] + curc[j])

    def store_fire(t, s):
        off = base + t * C
        pltpu.async_copy(obuf_v.at[s], out_hbm.at[pl.ds(off * D, C * D)],
                         ssems.at[s])

    NQ = NCHUNK // 4
    for t in range(4):
        fire_idx(t, t)
    gfire(0, 0, 0, drain_st=False)

    def quad(q, _):
        t0 = 4 * q
        for k in range(4):
            t = t0 + k
            ip = k
            s = k % 2
            if k < 3:
                gfire(t + 1, k + 1, (k + 1) % 2,
                      drain_st=True if k >= 1 else (q >= 1))
            else:
                @pl.when(q < NQ - 1)
                def _():
                    gfire(t + 1, 0, 0, drain_st=True)
            work_add(ip, s)

            @pl.when(q < NQ - 1)
            def _():
                fire_idx(t + 4, ip)
            store_fire(t, s)
        return 0

    lax.fori_loop(0, NQ, quad, 0)
    drain_store(0)
    drain_store(1)


def kernel(seq, seg, tok_table, seg_table, pos_table):
    seq_flat = seq.reshape(N).astype(jnp.int32)
    seg_flat = seg.reshape(N).astype(jnp.int32)
    mesh = plsc.VectorSubcoreMesh(core_axis_name="c", subcore_axis_name="s")
    run = pl.kernel(
        _embed_kernel,
        mesh=mesh,
        compiler_params=pltpu.CompilerParams(use_tc_tiling_on_sc=False),
        out_type=jax.ShapeDtypeStruct((N * D,), jnp.float32),
        scratch_types=[
            pltpu.VMEM((2, D), jnp.float32),
            pltpu.VMEM((L, D), jnp.float32),
            pltpu.VMEM((2 * L, D), jnp.float32),
            pltpu.VMEM((4, C), jnp.int32),
            pltpu.VMEM((4, C), jnp.int32),
            pltpu.VMEM((2, C, D), jnp.float32),
            pltpu.VMEM((2, C * D), jnp.float32),
            pltpu.SemaphoreType.DMA((2,)),
            pltpu.SemaphoreType.DMA((2,)),
            pltpu.SemaphoreType.DMA((4,)),
        ],
    )
    out = run(seq_flat, seg_flat, tok_table, seg_table, pos_table)
    return out.reshape(B, L, D)
